# baseline (speedup 1.0000x reference)
import functools

import jax
import jax.numpy as jnp
from jax import lax
from jax.experimental import pallas as pl
from jax.experimental.pallas import tpu as pltpu
from jax.experimental.pallas import tpu_sc as plsc

M = 1000000
N = 1000000
F = 64
B = 16384

_NC = 2
_NS = 16
_NW = _NC * _NS
_BPW = B // _NW
_IPW = _BPW * F
_NSTR = _IPW // 128


def _gather_body(idx_hbm, embT_flat, out_hbm, idx_v, flat_idx, rows, sem):
    wid = lax.axis_index("s") * _NC + lax.axis_index("c")
    base = wid * _BPW
    pltpu.sync_copy(idx_hbm.at[pl.ds(base, _BPW)], idx_v)

    lane = lax.iota(jnp.int32, 16)
    fM = [(lane + g * 16) * M for g in range(4)]

    def expand(jb, carry):
        vu = idx_v[pl.ds(jb * 16, 16)]
        for k in range(16):
            s = jb * 16 + k
            for g in range(4):
                flat_idx[pl.ds(s * F + g * 16, 16)] = vu[k] + fM[g]
        return carry

    lax.fori_loop(0, _BPW // 16, expand, 0)

    def fire(kb, carry):
        k = kb * 4
        copies = [
            pltpu.async_copy(embT_flat.at[flat_idx.at[pl.ds((k + t) * 128, 128)]],
                             rows.at[pl.ds((k + t) * 128, 128)], sem)
            for t in range(4)
        ]
        for c in copies:
            c.wait()
        return carry

    lax.fori_loop(0, _NSTR // 4, fire, 0)
    pltpu.sync_copy(rows, out_hbm.at[wid])


_gather_one = functools.partial(
    pl.kernel,
    mesh=plsc.VectorSubcoreMesh(core_axis_name="c", subcore_axis_name="s"),
    out_type=jax.ShapeDtypeStruct((_NW, _IPW), jnp.float32),
    scratch_types=[
        pltpu.VMEM((_BPW,), jnp.int32),
        pltpu.VMEM((_IPW,), jnp.int32),
        pltpu.VMEM((_IPW,), jnp.float32),
        pltpu.SemaphoreType.DMA,
    ],
    compiler_params=pltpu.CompilerParams(use_tc_tiling_on_sc=False),
)(_gather_body)


def _leaky(x):
    return jnp.where(x > 0, x, 0.01 * x)


def _mlp_body(u_ref, v_ref, w1u_ref, w1v_ref, b1_ref, w2_ref, b2_ref, o_ref):
    u = _leaky(u_ref[...])
    v = _leaky(v_ref[...])
    h = (jnp.dot(u, w1u_ref[...], preferred_element_type=jnp.float32)
         + jnp.dot(v, w1v_ref[...], preferred_element_type=jnp.float32)
         + b1_ref[...])
    h = _leaky(h)
    s = jnp.sum(h * w2_ref[...], axis=1, keepdims=True) + b2_ref[...]
    o_ref[...] = jnp.maximum(s, 0.0)


_BB = 2048


def _mlp(u, v, w1u, w1v, b1, w2r, b2):
    grid = (B // _BB,)
    return pl.pallas_call(
        _mlp_body,
        grid=grid,
        in_specs=[
            pl.BlockSpec((_BB, F), lambda i: (i, 0)),
            pl.BlockSpec((_BB, F), lambda i: (i, 0)),
            pl.BlockSpec((F, F), lambda i: (0, 0)),
            pl.BlockSpec((F, F), lambda i: (0, 0)),
            pl.BlockSpec((1, F), lambda i: (0, 0)),
            pl.BlockSpec((1, F), lambda i: (0, 0)),
            pl.BlockSpec((1, 1), lambda i: (0, 0)),
        ],
        out_specs=pl.BlockSpec((_BB, 1), lambda i: (i, 0)),
        out_shape=jax.ShapeDtypeStruct((B, 1), jnp.float32),
    )(u, v, w1u, w1v, b1, w2r, b2)


def kernel(users, items, user_emb, item_emb, W1, b1, W2, b2):
    u_rows = _gather_one(users.astype(jnp.int32),
                         user_emb.T.reshape(F * M)).reshape(B, F)
    v_rows = _gather_one(items.astype(jnp.int32),
                         item_emb.T.reshape(F * N)).reshape(B, F)
    return _mlp(u_rows, v_rows,
                W1[:F, :], W1[F:, :],
                b1.reshape(1, F), W2.reshape(1, F), b2.reshape(1, 1))


# speedup vs baseline: 8.9358x; 8.9358x over previous
import functools

import jax
import jax.numpy as jnp
from jax import lax
from jax.experimental import pallas as pl
from jax.experimental.pallas import tpu as pltpu
from jax.experimental.pallas import tpu_sc as plsc

M = 1000000
N = 1000000
F = 64
B = 16384

_NC = 2
_NS = 16
_NW = _NC * _NS
_BPW = B // _NW
_CH = 128
_NCH = _BPW // _CH


def _gather_body(idx_hbm, emb_hbm, out_hbm, idx_w, rows, sem):
    wid = lax.axis_index("s") * _NC + lax.axis_index("c")
    pltpu.sync_copy(idx_hbm.at[wid], idx_w)
    copies = []
    for j in range(_NCH):
        copies.append(pltpu.async_copy(
            emb_hbm.at[idx_w.at[j]], rows.at[pl.ds(j * _CH, _CH)], sem))
    for c in copies:
        c.wait()
    base = wid * _BPW
    pltpu.sync_copy(rows, out_hbm.at[pl.ds(base, _BPW)])


_gather_one = functools.partial(
    pl.kernel,
    mesh=plsc.VectorSubcoreMesh(core_axis_name="c", subcore_axis_name="s"),
    out_type=jax.ShapeDtypeStruct((B, F), jnp.float32),
    scratch_types=[
        pltpu.VMEM((_NCH, _CH), jnp.int32),
        pltpu.VMEM((_BPW, F), jnp.float32),
        pltpu.SemaphoreType.DMA,
    ],
    compiler_params=pltpu.CompilerParams(use_tc_tiling_on_sc=False),
)(_gather_body)


def _leaky(x):
    return jnp.where(x > 0, x, 0.01 * x)


def _mlp_body(u_ref, v_ref, w1u_ref, w1v_ref, b1_ref, w2_ref, b2_ref, o_ref):
    u = _leaky(u_ref[...])
    v = _leaky(v_ref[...])
    h = (jnp.dot(u, w1u_ref[...], preferred_element_type=jnp.float32)
         + jnp.dot(v, w1v_ref[...], preferred_element_type=jnp.float32)
         + b1_ref[...])
    h = _leaky(h)
    s = jnp.sum(h * w2_ref[...], axis=1, keepdims=True) + b2_ref[...]
    o_ref[...] = jnp.maximum(s, 0.0)


_BB = 2048


def _mlp(u, v, w1u, w1v, b1, w2r, b2):
    grid = (B // _BB,)
    return pl.pallas_call(
        _mlp_body,
        grid=grid,
        in_specs=[
            pl.BlockSpec((_BB, F), lambda i: (i, 0)),
            pl.BlockSpec((_BB, F), lambda i: (i, 0)),
            pl.BlockSpec((F, F), lambda i: (0, 0)),
            pl.BlockSpec((F, F), lambda i: (0, 0)),
            pl.BlockSpec((1, F), lambda i: (0, 0)),
            pl.BlockSpec((1, F), lambda i: (0, 0)),
            pl.BlockSpec((1, 1), lambda i: (0, 0)),
        ],
        out_specs=pl.BlockSpec((_BB, 1), lambda i: (i, 0)),
        out_shape=jax.ShapeDtypeStruct((B, 1), jnp.float32),
    )(u, v, w1u, w1v, b1, w2r, b2)


def kernel(users, items, user_emb, item_emb, W1, b1, W2, b2):
    users_r = users.astype(jnp.int32).reshape(_NW, _NCH, _CH)
    items_r = items.astype(jnp.int32).reshape(_NW, _NCH, _CH)
    u_rows = _gather_one(users_r, user_emb)
    v_rows = _gather_one(items_r, item_emb)
    return _mlp(u_rows, v_rows,
                W1[:F, :], W1[F:, :],
                b1.reshape(1, F), W2.reshape(1, F), b2.reshape(1, 1))
